# baseline (speedup 1.0000x reference)
import jax
import jax.numpy as jnp
from jax import lax
from jax.experimental import pallas as pl
from jax.experimental.pallas import tpu as pltpu
from jax.experimental.pallas import tpu_sc as plsc

ALPHA = 0.1
GAMMA = 0.99

M = 100000
A = 64
B = 16384

NC = 2
NS = 16
LANES = 16

HROWS = M // NC
CHUNKS = 2
CHUNK_ROWS = HROWS // CHUNKS
TB = B // NS
GCH = 128
NGCH = TB // GCH
VPG = GCH // LANES
VPR = A // LANES
HB = 128


def _body(q2d, sidx, nidx, act, rew, outbuf,
          sidx_v, nidx_v, act_v, rew_v, maxv_v,
          lrow2_v, rowredir2_v, delta2_v, rows_v, stage_v,
          semA, semB, semW,
          acc):
    c = lax.axis_index("c")
    s = lax.axis_index("s")
    iota = lax.iota(jnp.int32, LANES)

    bbase = s * TB
    pltpu.sync_copy(sidx.at[pl.ds(bbase, TB)], sidx_v)
    pltpu.sync_copy(nidx.at[pl.ds(bbase, TB)], nidx_v)
    pltpu.sync_copy(act.at[pl.ds(bbase, TB)], act_v)
    pltpu.sync_copy(rew.at[pl.ds(bbase, TB)], rew_v)

    G1 = 64
    NH = TB // G1

    def _gather_rows(idx_v, h, half, sem):
        d = pltpu.make_async_copy(
            q2d.at[idx_v.at[pl.ds(h * G1, G1)]],
            rows_v.at[pl.ds(half * G1, G1), :], sem)
        d.start()
        return d

    handles = [None, None]
    handles[0] = _gather_rows(nidx_v, 0, 0, semA)
    for h in range(NH):
        if h + 1 < NH:
            handles[(h + 1) % 2] = _gather_rows(nidx_v, h + 1, (h + 1) % 2,
                                                semA)
        handles[h % 2].wait()

        def _rowmax_body(g, _):
            rid = (h % 2) * G1 + g * LANES + iota

            def _col(c2, m):
                cid = jnp.full((LANES,), 0, jnp.int32) + c2
                return jnp.maximum(m, plsc.load_gather(rows_v, [rid, cid]))
            m = lax.fori_loop(0, A, _col,
                              jnp.full((LANES,), -jnp.inf, jnp.float32),
                              unroll=8)
            maxv_v[pl.ds(h * G1 + g * LANES, LANES)] = m
            return 0
        lax.fori_loop(0, G1 // LANES, _rowmax_body, 0)

    handles[0] = _gather_rows(sidx_v, 0, 0, semB)
    for h in range(NH):
        if h + 1 < NH:
            handles[(h + 1) % 2] = _gather_rows(sidx_v, h + 1, (h + 1) % 2,
                                                semB)
        handles[h % 2].wait()

        def _delta_body(g, _):
            sl = pl.ds(h * G1 + g * LANES, LANES)
            rid = (h % 2) * G1 + g * LANES + iota
            qs = plsc.load_gather(rows_v, [rid, act_v[sl]])
            maxv_v[sl] = ALPHA * (rew_v[sl] + GAMMA * maxv_v[sl] - qs)
            return 0
        lax.fori_loop(0, G1 // LANES, _delta_body, 0, unroll=4)

    def _zstage_body(r, _):
        for v in range(VPR):
            stage_v[r, pl.ds(v * LANES, LANES)] = (
                jnp.zeros((LANES,), jnp.float32))
        return 0
    lax.fori_loop(0, GCH, _zstage_body, 0, unroll=4)

    for k in range(CHUNKS):
        rb = (c * CHUNKS + k) * CHUNK_ROWS

        def _mask_body(i, _):
            sl = pl.ds(i * LANES, LANES)
            sr = sidx_v[sl]
            local = sr - rb
            inr = (local >= 0) & (local < CHUNK_ROWS)
            j = i // VPG
            l = i % VPG
            lsl = pl.ds(l * LANES, LANES)
            lrow2_v[j, lsl] = jnp.where(inr, local, 0)
            rowredir2_v[j, lsl] = jnp.where(inr, sr, rb)
            delta2_v[j, lsl] = jnp.where(inr, maxv_v[sl], 0.0)
            return 0
        lax.fori_loop(0, TB // LANES, _mask_body, 0, unroll=4)

        zh = []
        for j in range(NGCH):
            d = pltpu.make_async_copy(stage_v, acc.at[lrow2_v.at[j]], semA)
            d.start()
            zh.append(d)
        for d in zh:
            d.wait()
        plsc.subcore_barrier()

        G2 = 64
        NJ2 = TB // G2
        V2 = G2 // LANES

        def _fill(j, pp, zero):
            for l in range(V2):
                srow = pp * G2 + l * LANES + iota
                av = act_v[pl.ds(j * G2 + l * LANES, LANES)]
                if zero:
                    dv = jnp.zeros((LANES,), jnp.float32)
                else:
                    jj = (j * G2) // GCH
                    ll = (j * G2 % GCH) // LANES + l
                    dv = delta2_v[jj, pl.ds(ll * LANES, LANES)]
                plsc.store_scatter(stage_v, [srow, av], dv)

        ah = [None, None]
        for j in range(NJ2):
            pp = j % 2
            if j >= 2:
                ah[pp].wait()
                _fill(j - 2, pp, True)
            _fill(j, pp, False)
            lr = lrow2_v.at[(j * G2) // GCH, pl.ds(j * G2 % GCH, G2)]
            ah[pp] = pltpu.make_async_copy(
                stage_v.at[pl.ds(pp * G2, G2), :], acc.at[lr], semB)
            ah[pp].start(add=True)
        for j in (NJ2 - 2, NJ2 - 1):
            pp = j % 2
            ah[pp].wait()
            _fill(j, pp, True)
        plsc.subcore_barrier()

        G2r = 64
        NJr = TB // G2r
        gh = [None, None]
        wh = [None, None]

        def _fire_gathers(j, pp):
            lr = lrow2_v.at[(j * G2r) // GCH, pl.ds(j * G2r % GCH, G2r)]
            rr = rowredir2_v.at[(j * G2r) // GCH, pl.ds(j * G2r % GCH, G2r)]
            d0 = pltpu.make_async_copy(
                acc.at[lr], stage_v.at[pl.ds(pp * G2r, G2r), :], semA)
            d1 = pltpu.make_async_copy(
                q2d.at[rr], rows_v.at[pl.ds(pp * G2r, G2r), :], semB)
            d0.start()
            d1.start()
            return (d0, d1)

        gh[0] = _fire_gathers(0, 0)
        for j in range(NJr):
            pp = j % 2
            if j + 1 < NJr:
                if wh[1 - pp] is not None:
                    wh[1 - pp].wait()
                gh[1 - pp] = _fire_gathers(j + 1, 1 - pp)
            gh[pp][0].wait()
            gh[pp][1].wait()

            def _addrows_body(r, _):
                for v in range(VPR):
                    lsl = pl.ds(v * LANES, LANES)
                    stage_v[pp * G2r + r, lsl] = (
                        stage_v[pp * G2r + r, lsl]
                        + rows_v[pp * G2r + r, lsl])
                return 0
            lax.fori_loop(0, G2r, _addrows_body, 0, unroll=4)
            rr = rowredir2_v.at[(j * G2r) // GCH, pl.ds(j * G2r % GCH, G2r)]
            wh[pp] = pltpu.make_async_copy(
                stage_v.at[pl.ds(pp * G2r, G2r), :], outbuf.at[rr], semW)
            wh[pp].start()
        for d in wh:
            if d is not None:
                d.wait()

        def _rezero_body(r, _):
            for v in range(VPR):
                stage_v[r, pl.ds(v * LANES, LANES)] = (
                    jnp.zeros((LANES,), jnp.float32))
            return 0
        lax.fori_loop(0, 2 * G2r, _rezero_body, 0, unroll=4)

        plsc.subcore_barrier()


def _make_kernel():
    mesh = plsc.VectorSubcoreMesh(core_axis_name="c", subcore_axis_name="s")
    return pl.kernel(
        _body,
        out_type=(),
        mesh=mesh,
        compiler_params=pltpu.CompilerParams(
            needs_layout_passes=False, use_tc_tiling_on_sc=False),
        scratch_types=[
            pltpu.VMEM((TB,), jnp.int32),
            pltpu.VMEM((TB,), jnp.int32),
            pltpu.VMEM((TB,), jnp.int32),
            pltpu.VMEM((TB,), jnp.float32),
            pltpu.VMEM((TB,), jnp.float32),
            pltpu.VMEM((NGCH, GCH), jnp.int32),
            pltpu.VMEM((NGCH, GCH), jnp.int32),
            pltpu.VMEM((NGCH, GCH), jnp.float32),
            pltpu.VMEM((HB, A), jnp.float32),
            pltpu.VMEM((GCH, A), jnp.float32),
            pltpu.SemaphoreType.DMA,
            pltpu.SemaphoreType.DMA,
            pltpu.SemaphoreType.DMA,
            pltpu.VMEM_SHARED((CHUNK_ROWS, A), jnp.float32),
        ],
    )


@jax.jit
def _run(q_table, state_idx, next_state_idx, action, reward):
    outbuf = jax.new_ref(q_table)
    _make_kernel()(q_table, state_idx, next_state_idx, action, reward, outbuf)
    return outbuf[...]


def kernel(q_table, state_idx, next_state_idx, action, reward):
    return _run(q_table, state_idx, next_state_idx, action, reward)


# speedup vs baseline: 1.0043x; 1.0043x over previous
import jax
import jax.numpy as jnp
from jax import lax
from jax.experimental import pallas as pl
from jax.experimental.pallas import tpu as pltpu
from jax.experimental.pallas import tpu_sc as plsc

ALPHA = 0.1
GAMMA = 0.99

M = 100000
A = 64
B = 16384

NC = 2
NS = 16
LANES = 16

HROWS = M // NC
CHUNKS = 2
CHUNK_ROWS = HROWS // CHUNKS
TB = B // NS
GCH = 128
NGCH = TB // GCH
VPG = GCH // LANES
VPR = A // LANES
HB = 128


def _body(q2d, sidx, nidx, act, rew, outbuf,
          sidx_v, nidx_v, act_v, rew_v, maxv_v,
          lrow2_v, rowredir2_v, delta2_v, rows_v, stage_v,
          semA, semB, semW,
          acc):
    c = lax.axis_index("c")
    s = lax.axis_index("s")
    iota = lax.iota(jnp.int32, LANES)

    bbase = s * TB
    pltpu.sync_copy(sidx.at[pl.ds(bbase, TB)], sidx_v)
    pltpu.sync_copy(nidx.at[pl.ds(bbase, TB)], nidx_v)
    pltpu.sync_copy(act.at[pl.ds(bbase, TB)], act_v)
    pltpu.sync_copy(rew.at[pl.ds(bbase, TB)], rew_v)

    G1 = 64
    NH = TB // G1

    DEPTH = 4
    bufs = [(rows_v, 0), (rows_v, G1), (stage_v, 0), (stage_v, G1)]

    def _gather_rows(idx_v, h, sem):
        ref, off = bufs[h % DEPTH]
        d = pltpu.make_async_copy(
            q2d.at[idx_v.at[pl.ds(h * G1, G1)]],
            ref.at[pl.ds(off, G1), :], sem)
        d.start()
        return d

    handles = [None] * DEPTH
    for h in range(DEPTH - 1):
        handles[h] = _gather_rows(nidx_v, h, semA)
    for h in range(NH):
        if h + DEPTH - 1 < NH:
            handles[(h + DEPTH - 1) % DEPTH] = _gather_rows(
                nidx_v, h + DEPTH - 1, semA)
        handles[h % DEPTH].wait()
        ref, off = bufs[h % DEPTH]

        def _rowmax_body(g, _):
            rid = off + g * LANES + iota

            def _col(c2, m):
                cid = jnp.full((LANES,), 0, jnp.int32) + c2
                return jnp.maximum(m, plsc.load_gather(ref, [rid, cid]))
            m = lax.fori_loop(0, A, _col,
                              jnp.full((LANES,), -jnp.inf, jnp.float32),
                              unroll=8)
            maxv_v[pl.ds(h * G1 + g * LANES, LANES)] = m
            return 0
        lax.fori_loop(0, G1 // LANES, _rowmax_body, 0)

    for h in range(DEPTH - 1):
        handles[h] = _gather_rows(sidx_v, h, semB)
    for h in range(NH):
        if h + DEPTH - 1 < NH:
            handles[(h + DEPTH - 1) % DEPTH] = _gather_rows(
                sidx_v, h + DEPTH - 1, semB)
        handles[h % DEPTH].wait()
        ref, off = bufs[h % DEPTH]

        def _delta_body(g, _):
            sl = pl.ds(h * G1 + g * LANES, LANES)
            rid = off + g * LANES + iota
            qs = plsc.load_gather(ref, [rid, act_v[sl]])
            maxv_v[sl] = ALPHA * (rew_v[sl] + GAMMA * maxv_v[sl] - qs)
            return 0
        lax.fori_loop(0, G1 // LANES, _delta_body, 0, unroll=4)

    def _zstage_body(r, _):
        for v in range(VPR):
            stage_v[r, pl.ds(v * LANES, LANES)] = (
                jnp.zeros((LANES,), jnp.float32))
        return 0
    lax.fori_loop(0, GCH, _zstage_body, 0, unroll=4)

    for k in range(CHUNKS):
        rb = (c * CHUNKS + k) * CHUNK_ROWS

        def _mask_body(i, _):
            sl = pl.ds(i * LANES, LANES)
            sr = sidx_v[sl]
            local = sr - rb
            inr = (local >= 0) & (local < CHUNK_ROWS)
            j = i // VPG
            l = i % VPG
            lsl = pl.ds(l * LANES, LANES)
            lrow2_v[j, lsl] = jnp.where(inr, local, 0)
            rowredir2_v[j, lsl] = jnp.where(inr, sr, rb)
            delta2_v[j, lsl] = jnp.where(inr, maxv_v[sl], 0.0)
            return 0
        lax.fori_loop(0, TB // LANES, _mask_body, 0, unroll=4)

        zh = []
        for j in range(NGCH):
            d = pltpu.make_async_copy(stage_v, acc.at[lrow2_v.at[j]], semA)
            d.start()
            zh.append(d)
        for d in zh:
            d.wait()
        plsc.subcore_barrier()

        G2 = 64
        NJ2 = TB // G2
        V2 = G2 // LANES

        def _fill(j, pp, zero):
            for l in range(V2):
                srow = pp * G2 + l * LANES + iota
                av = act_v[pl.ds(j * G2 + l * LANES, LANES)]
                if zero:
                    dv = jnp.zeros((LANES,), jnp.float32)
                else:
                    jj = (j * G2) // GCH
                    ll = (j * G2 % GCH) // LANES + l
                    dv = delta2_v[jj, pl.ds(ll * LANES, LANES)]
                plsc.store_scatter(stage_v, [srow, av], dv)

        ah = [None, None]
        for j in range(NJ2):
            pp = j % 2
            if j >= 2:
                ah[pp].wait()
                _fill(j - 2, pp, True)
            _fill(j, pp, False)
            lr = lrow2_v.at[(j * G2) // GCH, pl.ds(j * G2 % GCH, G2)]
            ah[pp] = pltpu.make_async_copy(
                stage_v.at[pl.ds(pp * G2, G2), :], acc.at[lr], semB)
            ah[pp].start(add=True)
        for j in (NJ2 - 2, NJ2 - 1):
            pp = j % 2
            ah[pp].wait()
            _fill(j, pp, True)
        plsc.subcore_barrier()

        G2r = 64
        NJr = TB // G2r
        gh = [None, None]
        wh = [None, None]

        def _fire_gathers(j, pp):
            lr = lrow2_v.at[(j * G2r) // GCH, pl.ds(j * G2r % GCH, G2r)]
            rr = rowredir2_v.at[(j * G2r) // GCH, pl.ds(j * G2r % GCH, G2r)]
            d0 = pltpu.make_async_copy(
                acc.at[lr], stage_v.at[pl.ds(pp * G2r, G2r), :], semA)
            d1 = pltpu.make_async_copy(
                q2d.at[rr], rows_v.at[pl.ds(pp * G2r, G2r), :], semB)
            d0.start()
            d1.start()
            return (d0, d1)

        gh[0] = _fire_gathers(0, 0)
        for j in range(NJr):
            pp = j % 2
            if j + 1 < NJr:
                if wh[1 - pp] is not None:
                    wh[1 - pp].wait()
                gh[1 - pp] = _fire_gathers(j + 1, 1 - pp)
            gh[pp][0].wait()
            gh[pp][1].wait()

            def _addrows_body(r, _):
                for v in range(VPR):
                    lsl = pl.ds(v * LANES, LANES)
                    stage_v[pp * G2r + r, lsl] = (
                        stage_v[pp * G2r + r, lsl]
                        + rows_v[pp * G2r + r, lsl])
                return 0
            lax.fori_loop(0, G2r, _addrows_body, 0, unroll=4)
            rr = rowredir2_v.at[(j * G2r) // GCH, pl.ds(j * G2r % GCH, G2r)]
            wh[pp] = pltpu.make_async_copy(
                stage_v.at[pl.ds(pp * G2r, G2r), :], outbuf.at[rr], semW)
            wh[pp].start()
        for d in wh:
            if d is not None:
                d.wait()

        def _rezero_body(r, _):
            for v in range(VPR):
                stage_v[r, pl.ds(v * LANES, LANES)] = (
                    jnp.zeros((LANES,), jnp.float32))
            return 0
        lax.fori_loop(0, 2 * G2r, _rezero_body, 0, unroll=4)

        plsc.subcore_barrier()


def _make_kernel():
    mesh = plsc.VectorSubcoreMesh(core_axis_name="c", subcore_axis_name="s")
    return pl.kernel(
        _body,
        out_type=(),
        mesh=mesh,
        compiler_params=pltpu.CompilerParams(
            needs_layout_passes=False, use_tc_tiling_on_sc=False),
        scratch_types=[
            pltpu.VMEM((TB,), jnp.int32),
            pltpu.VMEM((TB,), jnp.int32),
            pltpu.VMEM((TB,), jnp.int32),
            pltpu.VMEM((TB,), jnp.float32),
            pltpu.VMEM((TB,), jnp.float32),
            pltpu.VMEM((NGCH, GCH), jnp.int32),
            pltpu.VMEM((NGCH, GCH), jnp.int32),
            pltpu.VMEM((NGCH, GCH), jnp.float32),
            pltpu.VMEM((HB, A), jnp.float32),
            pltpu.VMEM((GCH, A), jnp.float32),
            pltpu.SemaphoreType.DMA,
            pltpu.SemaphoreType.DMA,
            pltpu.SemaphoreType.DMA,
            pltpu.VMEM_SHARED((CHUNK_ROWS, A), jnp.float32),
        ],
    )


@jax.jit
def _run(q_table, state_idx, next_state_idx, action, reward):
    outbuf = jax.new_ref(q_table)
    _make_kernel()(q_table, state_idx, next_state_idx, action, reward, outbuf)
    return outbuf[...]


def kernel(q_table, state_idx, next_state_idx, action, reward):
    return _run(q_table, state_idx, next_state_idx, action, reward)
